# Initial kernel scaffold; baseline (speedup 1.0000x reference)
import jax
import jax.numpy as jnp
from jax.experimental import pallas as pl


def kernel(feat, turn, control, intersect, ctrs, actors, actor_ctrs, idcs, actor_idcs, meta_w, meta_gw, meta_gb, b0_dist_w1, b0_dist_b1, b0_dist_w2, b0_dist_gw, b0_dist_gb, b0_query_w, b0_query_gw, b0_query_gb, b0_ctx_w1, b0_ctx_gw, b0_ctx_gb, b0_ctx_w2, b0_agt_w, b0_norm_w, b0_norm_b, b0_lin_w, b0_lin_gw, b0_lin_gb, b1_dist_w1, b1_dist_b1, b1_dist_w2, b1_dist_gw, b1_dist_gb, b1_query_w, b1_query_gw, b1_query_gb, b1_ctx_w1, b1_ctx_gw, b1_ctx_gb, b1_ctx_w2, b1_agt_w, b1_norm_w, b1_norm_b, b1_lin_w, b1_lin_gw, b1_lin_gb):
    raise NotImplementedError("write your pallas kernel here")


# speedup vs baseline: 3.2136x; 3.2136x over previous
import functools
import jax
import jax.numpy as jnp
from jax.experimental import pallas as pl
from jax.experimental.pallas import tpu as pltpu

D = 128
N_MAP = 10000
N_ACT = 1000
DIST_TH = 7.0
K = 64
M = 128
KC = 16
NPAD = 10240


def _gn(x, w, b):
    mu = jnp.mean(x, axis=-1, keepdims=True)
    var = jnp.mean((x - mu) ** 2, axis=-1, keepdims=True)
    return (x - mu) * jax.lax.rsqrt(var + 1e-5) * w + b


def _mlp_kernel(feat_ref, meta8_ref, cnt_ref, dxy_ref, ag_ref, vrow_ref,
                mwf_ref, mwm_ref,
                w1_0_ref, dw2_0_ref, qw_0_ref, wq_0_ref, ag_0_ref, wd_0_ref,
                wf_0_ref, cw2_0_ref, lin_0_ref,
                w1_1_ref, dw2_1_ref, qw_1_ref, wq_1_ref, ag_1_ref, wd_1_ref,
                wf_1_ref, cw2_1_ref, lin_1_ref,
                out_ref):
    v = vrow_ref[...]

    def row(i):
        return v[i][None, :]

    feat = feat_ref[...]
    meta8 = meta8_ref[...]
    cnt = cnt_ref[...]

    x = feat @ mwf_ref[...] + meta8 @ mwm_ref[...]
    x = jax.nn.relu(_gn(x, row(0), row(1)))

    iota3 = jax.lax.broadcasted_iota(jnp.int32, (M, KC, D), 1)

    blk = ((w1_0_ref, dw2_0_ref, qw_0_ref, wq_0_ref, ag_0_ref, wd_0_ref,
            wf_0_ref, cw2_0_ref, lin_0_ref),
           (w1_1_ref, dw2_1_ref, qw_1_ref, wq_1_ref, ag_1_ref, wd_1_ref,
            wf_1_ref, cw2_1_ref, lin_1_ref))

    for b in range(2):
        w1, dw2, qw, wq, agw, wd, wf, cw2, lin = blk[b]
        base = 2 + 11 * b
        q = jax.nn.relu(_gn(x @ qw[...], row(base + 3), row(base + 4)))
        qp = q @ wq[...]
        acc = x @ agw[...]

        w1m = w1[...]
        dw2m = dw2[...]
        wdm = wd[...]
        wfm = wf[...]
        cw2m = cw2[...]
        b1 = row(base + 0)
        dgw, dgb = row(base + 1), row(base + 2)
        cgw, cgb = row(base + 5), row(base + 6)

        for s in range(K // KC):
            dxy = dxy_ref[:, s * KC:(s + 1) * KC, :].reshape(M * KC, 8)
            agt = ag_ref[:, s * KC:(s + 1) * KC, :].reshape(M * KC, D)
            d1 = jax.nn.relu(dxy @ w1m + b1)
            d2 = jax.nn.relu(_gn(d1 @ dw2m, dgw, dgb))
            h = d2 @ wdm + agt @ wfm
            h = h.reshape(M, KC, D) + qp[:, None, :]
            h = jax.nn.relu(_gn(h, cgw[None], cgb[None]))
            c = h.reshape(M * KC, D) @ cw2m
            c = c.reshape(M, KC, D)
            valid = (iota3 + s * KC) < cnt[:, :, None]
            acc = acc + jnp.sum(jnp.where(valid, c, 0.0), axis=1)

        a = jax.nn.relu(_gn(acc, row(base + 7), row(base + 8)))
        a = _gn(a @ lin[...], row(base + 9), row(base + 10))
        x = jax.nn.relu(a + x)

    out_ref[...] = x


def _run_mlp(feat_p, meta8, cnt2, dxy8, ag, vrow, mats):
    grid = (NPAD // M,)
    bs_w = lambda shape: pl.BlockSpec(shape, lambda g: (0,) * len(shape))
    in_specs = [
        pl.BlockSpec((M, D), lambda g: (g, 0)),
        pl.BlockSpec((M, 8), lambda g: (g, 0)),
        pl.BlockSpec((M, 1), lambda g: (g, 0)),
        pl.BlockSpec((M, K, 8), lambda g: (g, 0, 0)),
        pl.BlockSpec((M, K, D), lambda g: (g, 0, 0)),
        bs_w(vrow.shape),
    ] + [bs_w(m.shape) for m in mats]
    return pl.pallas_call(
        _mlp_kernel,
        grid=grid,
        in_specs=in_specs,
        out_specs=pl.BlockSpec((M, D), lambda g: (g, 0)),
        out_shape=jax.ShapeDtypeStruct((NPAD, D), jnp.float32),
    )(feat_p, meta8, cnt2, dxy8, ag, vrow, *mats)


def kernel(feat, turn, control, intersect, ctrs, actors, actor_ctrs, idcs,
           actor_idcs, meta_w, meta_gw, meta_gb,
           b0_dist_w1, b0_dist_b1, b0_dist_w2, b0_dist_gw, b0_dist_gb,
           b0_query_w, b0_query_gw, b0_query_gb,
           b0_ctx_w1, b0_ctx_gw, b0_ctx_gb, b0_ctx_w2,
           b0_agt_w, b0_norm_w, b0_norm_b,
           b0_lin_w, b0_lin_gw, b0_lin_gb,
           b1_dist_w1, b1_dist_b1, b1_dist_w2, b1_dist_gw, b1_dist_gb,
           b1_query_w, b1_query_gw, b1_query_gb,
           b1_ctx_w1, b1_ctx_gw, b1_ctx_gb, b1_ctx_w2,
           b1_agt_w, b1_norm_w, b1_norm_b,
           b1_lin_w, b1_lin_gw, b1_lin_gb):
    d2 = jnp.sum((ctrs[:, None, :] - actor_ctrs[None, :, :]) ** 2, axis=-1)
    mask = d2 <= DIST_TH * DIST_TH
    cnt = jnp.sum(mask, axis=1).astype(jnp.int32)
    order = jnp.argsort(~mask, axis=1, stable=True)[:, :K]
    slot_iota = jnp.arange(K, dtype=jnp.int32)[None, :]
    nbr = jnp.where(slot_iota < cnt[:, None], order, 0).astype(jnp.int32)

    ag = actors[nbr]
    dx = ctrs[:, 0:1] - actor_ctrs[nbr, 0]
    dy = ctrs[:, 1:2] - actor_ctrs[nbr, 1]
    dxy8 = jnp.zeros((N_MAP, K, 8), jnp.float32)
    dxy8 = dxy8.at[:, :, 0].set(dx).at[:, :, 1].set(dy)

    pad = NPAD - N_MAP
    feat_p = jnp.pad(feat, ((0, pad), (0, 0)))
    meta = jnp.concatenate([turn, control[:, None], intersect[:, None]],
                           axis=1)
    meta8 = jnp.pad(meta, ((0, pad), (0, 4)))
    cnt2 = jnp.pad(cnt[:, None], ((0, pad), (0, 0)))
    dxy8 = jnp.pad(dxy8, ((0, pad), (0, 0), (0, 0)))
    ag = jnp.pad(ag, ((0, pad), (0, 0), (0, 0)))

    vrow = jnp.stack(
        [meta_gw, meta_gb,
         b0_dist_b1, b0_dist_gw, b0_dist_gb, b0_query_gw, b0_query_gb,
         b0_ctx_gw, b0_ctx_gb, b0_norm_w, b0_norm_b, b0_lin_gw, b0_lin_gb,
         b1_dist_b1, b1_dist_gw, b1_dist_gb, b1_query_gw, b1_query_gb,
         b1_ctx_gw, b1_ctx_gb, b1_norm_w, b1_norm_b, b1_lin_gw, b1_lin_gb])

    mwf = meta_w[:, :D].T
    mwm = jnp.pad(meta_w[:, D:].T, ((0, 4), (0, 0)))

    def blk_mats(dist_w1, dist_w2, query_w, ctx_w1, ctx_w2, agt_w, lin_w):
        w1 = jnp.pad(dist_w1.T, ((0, 6), (0, 0)))
        return (w1, dist_w2.T, query_w.T, ctx_w1[:, D:2 * D].T, agt_w.T,
                ctx_w1[:, :D].T, ctx_w1[:, 2 * D:].T, ctx_w2.T, lin_w.T)

    mats = ((mwf, mwm)
            + blk_mats(b0_dist_w1, b0_dist_w2, b0_query_w, b0_ctx_w1,
                       b0_ctx_w2, b0_agt_w, b0_lin_w)
            + blk_mats(b1_dist_w1, b1_dist_w2, b1_query_w, b1_ctx_w1,
                       b1_ctx_w2, b1_agt_w, b1_lin_w))

    out = _run_mlp(feat_p, meta8, cnt2, dxy8, ag, vrow, list(mats))
    return out[:N_MAP]


# speedup vs baseline: 3.9667x; 1.2343x over previous
import functools
import jax
import jax.numpy as jnp
from jax import lax
from jax.experimental import pallas as pl
from jax.experimental.pallas import tpu as pltpu
from jax.experimental.pallas import tpu_sc as plsc

D = 128
N_MAP = 10000
N_ACT = 1000
DIST_TH = 7.0
K = 64
M = 128
KC = 16
NPAD = 10240
NA_PAD = 1008
NW = 32
NPT = NPAD // NW


def _sc_build(cx_hbm, cy_hbm, ax_hbm, ay_hbm, actors_hbm,
              cnt_hbm, dxy_hbm, ag_hbm,
              axv, ayv, cxv, cyv, nbrv, dxyv, agv, cntv, sem):
    wid = lax.axis_index("s") * 2 + lax.axis_index("c")
    base = wid * NPT
    pltpu.sync_copy(ax_hbm, axv)
    pltpu.sync_copy(ay_hbm, ayv)
    pltpu.sync_copy(cx_hbm.at[pl.ds(base, NPT)], cxv)
    pltpu.sync_copy(cy_hbm.at[pl.ds(base, NPT)], cyv)
    lanes = lax.iota(jnp.int32, 16)
    zeros16 = jnp.zeros((16,), jnp.int32)
    th2 = DIST_TH * DIST_TH

    def node_body(i, _):
        isplat = jnp.zeros((16,), jnp.int32) + i
        cxi = plsc.load_gather(cxv, [isplat])
        cyi = plsc.load_gather(cyv, [isplat])
        for t in range(K // 16):
            nbrv[pl.ds(t * 16, 16)] = zeros16

        def chunk(jc, cnt_n):
            j0 = pl.multiple_of(jc * 16, 16)
            dxl = cxi - axv[pl.ds(j0, 16)]
            dyl = cyi - ayv[pl.ds(j0, 16)]
            m = (dxl * dxl + dyl * dyl) <= th2
            mi = m.astype(jnp.int32)
            pos = cnt_n + plsc.cumsum(mi) - 1
            ok = jnp.logical_and(m, pos < K)
            posc = jnp.minimum(pos, K - 1)
            plsc.store_scatter(nbrv, [posc], lanes + j0, mask=ok)
            plsc.store_scatter(dxyv, [posc * 8], dxl, mask=ok)
            plsc.store_scatter(dxyv, [posc * 8 + 1], dyl, mask=ok)
            return cnt_n + jnp.sum(mi)

        cnt_n = lax.fori_loop(0, NA_PAD // 16, chunk, jnp.int32(0))
        cntk = jnp.minimum(cnt_n, K)
        plsc.store_scatter(cntv, [jnp.zeros((16,), jnp.int32) + i],
                           jnp.zeros((16,), jnp.int32) + cntk,
                           mask=lanes == 0)
        pltpu.async_copy(actors_hbm.at[nbrv], agv, sem).wait()
        row = pl.multiple_of((base + i) * K, 64)
        pltpu.sync_copy(agv, ag_hbm.at[pl.ds(row, K)])
        off = pl.multiple_of((base + i) * K * 8, 512)
        pltpu.sync_copy(dxyv, dxy_hbm.at[pl.ds(off, K * 8)])
        return 0

    lax.fori_loop(0, NPT, node_body, 0)
    pltpu.sync_copy(cntv, cnt_hbm.at[pl.ds(base, NPT)])


def _run_sc_build(cx, cy, ax, ay, actors):
    mesh = plsc.VectorSubcoreMesh(core_axis_name="c", subcore_axis_name="s")
    f = pl.kernel(
        _sc_build,
        out_type=(jax.ShapeDtypeStruct((NPAD,), jnp.int32),
                  jax.ShapeDtypeStruct((NPAD * K * 8,), jnp.float32),
                  jax.ShapeDtypeStruct((NPAD * K, D), jnp.float32)),
        mesh=mesh,
        compiler_params=pltpu.CompilerParams(needs_layout_passes=False),
        scratch_types=[
            pltpu.VMEM((NA_PAD,), jnp.float32),
            pltpu.VMEM((NA_PAD,), jnp.float32),
            pltpu.VMEM((NPT,), jnp.float32),
            pltpu.VMEM((NPT,), jnp.float32),
            pltpu.VMEM((K,), jnp.int32),
            pltpu.VMEM((K * 8,), jnp.float32),
            pltpu.VMEM((K, D), jnp.float32),
            pltpu.VMEM((NPT,), jnp.int32),
            pltpu.SemaphoreType.DMA,
        ],
    )
    return f(cx, cy, ax, ay, actors)


def _gn(x, w, b):
    mu = jnp.mean(x, axis=-1, keepdims=True)
    var = jnp.mean((x - mu) ** 2, axis=-1, keepdims=True)
    return (x - mu) * jax.lax.rsqrt(var + 1e-5) * w + b


def _mlp_kernel(feat_ref, meta8_ref, cnt_ref, dxy_ref, ag_ref, vrow_ref,
                mwf_ref, mwm_ref,
                w1_0_ref, dw2_0_ref, qw_0_ref, wq_0_ref, ag_0_ref, wd_0_ref,
                wf_0_ref, cw2_0_ref, lin_0_ref,
                w1_1_ref, dw2_1_ref, qw_1_ref, wq_1_ref, ag_1_ref, wd_1_ref,
                wf_1_ref, cw2_1_ref, lin_1_ref,
                out_ref):
    v = vrow_ref[...]

    def row(i):
        return v[i][None, :]

    feat = feat_ref[...]
    meta8 = meta8_ref[...]
    cnt = cnt_ref[...]

    x = feat @ mwf_ref[...] + meta8 @ mwm_ref[...]
    x = jax.nn.relu(_gn(x, row(0), row(1)))

    iota3 = jax.lax.broadcasted_iota(jnp.int32, (M, KC, D), 1)

    blk = ((w1_0_ref, dw2_0_ref, qw_0_ref, wq_0_ref, ag_0_ref, wd_0_ref,
            wf_0_ref, cw2_0_ref, lin_0_ref),
           (w1_1_ref, dw2_1_ref, qw_1_ref, wq_1_ref, ag_1_ref, wd_1_ref,
            wf_1_ref, cw2_1_ref, lin_1_ref))

    for b in range(2):
        w1, dw2, qw, wq, agw, wd, wf, cw2, lin = blk[b]
        base = 2 + 11 * b
        q = jax.nn.relu(_gn(x @ qw[...], row(base + 3), row(base + 4)))
        qp = q @ wq[...]
        acc = x @ agw[...]

        w1m = w1[...]
        dw2m = dw2[...]
        wdm = wd[...]
        wfm = wf[...]
        cw2m = cw2[...]
        b1 = row(base + 0)
        dgw, dgb = row(base + 1), row(base + 2)
        cgw, cgb = row(base + 5), row(base + 6)

        for s in range(K // KC):
            dxy = dxy_ref[:, s * KC:(s + 1) * KC, :].reshape(M * KC, 8)
            agt = ag_ref[:, s * KC:(s + 1) * KC, :].reshape(M * KC, D)
            d1 = jax.nn.relu(dxy @ w1m + b1)
            d2 = jax.nn.relu(_gn(d1 @ dw2m, dgw, dgb))
            h = d2 @ wdm + agt @ wfm
            h = h.reshape(M, KC, D) + qp[:, None, :]
            h = jax.nn.relu(_gn(h, cgw[None], cgb[None]))
            c = h.reshape(M * KC, D) @ cw2m
            c = c.reshape(M, KC, D)
            valid = (iota3 + s * KC) < cnt[:, :, None]
            acc = acc + jnp.sum(jnp.where(valid, c, 0.0), axis=1)

        a = jax.nn.relu(_gn(acc, row(base + 7), row(base + 8)))
        a = _gn(a @ lin[...], row(base + 9), row(base + 10))
        x = jax.nn.relu(a + x)

    out_ref[...] = x


def _run_mlp(feat_p, meta8, cnt2, dxy8, ag, vrow, mats):
    grid = (NPAD // M,)
    bs_w = lambda shape: pl.BlockSpec(shape, lambda g: (0,) * len(shape))
    in_specs = [
        pl.BlockSpec((M, D), lambda g: (g, 0)),
        pl.BlockSpec((M, 8), lambda g: (g, 0)),
        pl.BlockSpec((M, 1), lambda g: (g, 0)),
        pl.BlockSpec((M, K, 8), lambda g: (g, 0, 0)),
        pl.BlockSpec((M, K, D), lambda g: (g, 0, 0)),
        bs_w(vrow.shape),
    ] + [bs_w(m.shape) for m in mats]
    return pl.pallas_call(
        _mlp_kernel,
        grid=grid,
        in_specs=in_specs,
        out_specs=pl.BlockSpec((M, D), lambda g: (g, 0)),
        out_shape=jax.ShapeDtypeStruct((NPAD, D), jnp.float32),
    )(feat_p, meta8, cnt2, dxy8, ag, vrow, *mats)


def kernel(feat, turn, control, intersect, ctrs, actors, actor_ctrs, idcs,
           actor_idcs, meta_w, meta_gw, meta_gb,
           b0_dist_w1, b0_dist_b1, b0_dist_w2, b0_dist_gw, b0_dist_gb,
           b0_query_w, b0_query_gw, b0_query_gb,
           b0_ctx_w1, b0_ctx_gw, b0_ctx_gb, b0_ctx_w2,
           b0_agt_w, b0_norm_w, b0_norm_b,
           b0_lin_w, b0_lin_gw, b0_lin_gb,
           b1_dist_w1, b1_dist_b1, b1_dist_w2, b1_dist_gw, b1_dist_gb,
           b1_query_w, b1_query_gw, b1_query_gb,
           b1_ctx_w1, b1_ctx_gw, b1_ctx_gb, b1_ctx_w2,
           b1_agt_w, b1_norm_w, b1_norm_b,
           b1_lin_w, b1_lin_gw, b1_lin_gb):
    pad = NPAD - N_MAP
    apad = NA_PAD - N_ACT
    cx = jnp.pad(ctrs[:, 0], (0, pad), constant_values=1e6)
    cy = jnp.pad(ctrs[:, 1], (0, pad), constant_values=1e6)
    ax = jnp.pad(actor_ctrs[:, 0], (0, apad), constant_values=-1e6)
    ay = jnp.pad(actor_ctrs[:, 1], (0, apad), constant_values=-1e6)
    cnt, dxy, ag = _run_sc_build(cx, cy, ax, ay, actors)
    cnt2 = cnt[:, None]
    dxy8 = dxy.reshape(NPAD, K, 8)
    ag = ag.reshape(NPAD, K, D)

    feat_p = jnp.pad(feat, ((0, pad), (0, 0)))
    meta = jnp.concatenate([turn, control[:, None], intersect[:, None]],
                           axis=1)
    meta8 = jnp.pad(meta, ((0, pad), (0, 4)))

    vrow = jnp.stack(
        [meta_gw, meta_gb,
         b0_dist_b1, b0_dist_gw, b0_dist_gb, b0_query_gw, b0_query_gb,
         b0_ctx_gw, b0_ctx_gb, b0_norm_w, b0_norm_b, b0_lin_gw, b0_lin_gb,
         b1_dist_b1, b1_dist_gw, b1_dist_gb, b1_query_gw, b1_query_gb,
         b1_ctx_gw, b1_ctx_gb, b1_norm_w, b1_norm_b, b1_lin_gw, b1_lin_gb])

    mwf = meta_w[:, :D].T
    mwm = jnp.pad(meta_w[:, D:].T, ((0, 4), (0, 0)))

    def blk_mats(dist_w1, dist_w2, query_w, ctx_w1, ctx_w2, agt_w, lin_w):
        w1 = jnp.pad(dist_w1.T, ((0, 6), (0, 0)))
        return (w1, dist_w2.T, query_w.T, ctx_w1[:, D:2 * D].T, agt_w.T,
                ctx_w1[:, :D].T, ctx_w1[:, 2 * D:].T, ctx_w2.T, lin_w.T)

    mats = ((mwf, mwm)
            + blk_mats(b0_dist_w1, b0_dist_w2, b0_query_w, b0_ctx_w1,
                       b0_ctx_w2, b0_agt_w, b0_lin_w)
            + blk_mats(b1_dist_w1, b1_dist_w2, b1_query_w, b1_ctx_w1,
                       b1_ctx_w2, b1_agt_w, b1_lin_w))

    out = _run_mlp(feat_p, meta8, cnt2, dxy8, ag, vrow, list(mats))
    return out[:N_MAP]
